# baseline (speedup 1.0000x reference)
import functools

import jax
import jax.numpy as jnp
from jax.experimental import pallas as pl
from jax.experimental.pallas import tpu as pltpu


def _build_x3(x_ref, x3_ref):
    Cin = x_ref.shape[1]
    HW = x_ref.shape[2] * x_ref.shape[3] * x_ref.shape[4]
    HPW = x3_ref.shape[1]
    W = (HPW - HW) // 2

    x = x_ref[0].astype(jnp.bfloat16).reshape(Cin, HW)

    lane = jax.lax.broadcasted_iota(jnp.int32, (Cin, HW), 1) % W
    left = jnp.concatenate([x[:, :1], x[:, :-1]], axis=1)
    right = jnp.concatenate([x[:, 1:], x[:, -1:]], axis=1)
    xl = jnp.where(lane == 0, right, left)
    xr = jnp.where(lane == W - 1, left, right)

    for i, vb in enumerate((xl, x, xr)):
        r0 = i * Cin
        x3_ref[r0:r0 + Cin, W:W + HW] = vb
        x3_ref[r0:r0 + Cin, 0:W] = vb[:, W:2 * W]
        x3_ref[r0:r0 + Cin, W + HW:HPW] = vb[:, HW - 2 * W:HW - W]


def _conv_acc(w_ref, b_ref, x3_ref, HW, W):
    acc = None
    for dy in range(3):
        contrib = jnp.dot(w_ref[dy], x3_ref[:, dy * W:dy * W + HW],
                          preferred_element_type=jnp.float32)
        acc = contrib if acc is None else acc + contrib
    return acc + b_ref[...]


def _stats_kernel(x_ref, w_ref, b_ref, st_ref, x3_ref):
    HW = x_ref.shape[2] * x_ref.shape[3] * x_ref.shape[4]
    W = (x3_ref.shape[1] - HW) // 2
    _build_x3(x_ref, x3_ref)
    acc = _conv_acc(w_ref, b_ref, x3_ref, HW, W)
    s = jnp.sum(acc, axis=1, keepdims=True)
    ss = jnp.sum(acc * acc, axis=1, keepdims=True)
    st_ref[0] = jnp.concatenate([s, ss], axis=1)


def _conv_bn_relu_kernel(x_ref, w_ref, b_ref, st_ref, g_ref, be_ref, o_ref,
                         x3_ref, *, eps, cnt):
    HW = x_ref.shape[2] * x_ref.shape[3] * x_ref.shape[4]
    W = (x3_ref.shape[1] - HW) // 2
    _build_x3(x_ref, x3_ref)
    acc = _conv_acc(w_ref, b_ref, x3_ref, HW, W)

    st = jnp.sum(st_ref[...], axis=0)
    mean = st[:, 0:1] / cnt
    var = jnp.maximum(st[:, 1:2] / cnt - mean * mean, 0.0)
    scale = g_ref[...] * jax.lax.rsqrt(var + eps)
    shift = be_ref[...] - mean * scale

    z = jnp.maximum(acc * scale + shift, 0.0)
    o_ref[0] = z.reshape(o_ref.shape[1:])


def kernel(x_nchw, weight, bias, gamma, beta):
    eps = 1e-5
    x = x_nchw.astype(jnp.float32)
    N, Cin, H, W = x.shape
    Cout = weight.shape[0]
    HW = H * W
    HPW = (H + 2) * W
    H8 = H // 8

    xf = x.reshape(N, Cin, H8, 8, W)
    w_r = (jnp.transpose(weight.astype(jnp.float32), (2, 0, 3, 1))
           .reshape(3, Cout, 3 * Cin).astype(jnp.bfloat16))
    b2 = bias.astype(jnp.float32).reshape(Cout, 1)
    g2 = gamma.astype(jnp.float32).reshape(Cout, 1)
    be2 = beta.astype(jnp.float32).reshape(Cout, 1)

    st = pl.pallas_call(
        _stats_kernel,
        out_shape=jax.ShapeDtypeStruct((N, Cout, 2), jnp.float32),
        name="conv_stats",
        grid=(N,),
        in_specs=[pl.BlockSpec((1, Cin, H8, 8, W), lambda g: (g, 0, 0, 0, 0)),
                  pl.BlockSpec((3, Cout, 3 * Cin), lambda g: (0, 0, 0)),
                  pl.BlockSpec((Cout, 1), lambda g: (0, 0))],
        out_specs=pl.BlockSpec((1, Cout, 2), lambda g: (g, 0, 0)),
        scratch_shapes=[pltpu.VMEM((3 * Cin, HPW), jnp.bfloat16)],
        compiler_params=pltpu.CompilerParams(
            dimension_semantics=("parallel",),
            vmem_limit_bytes=64 * 1024 * 1024),
    )(xf, w_r, b2)

    out = pl.pallas_call(
        functools.partial(_conv_bn_relu_kernel, eps=eps, cnt=float(N * HW)),
        out_shape=jax.ShapeDtypeStruct((N, Cout, H8, 8, W), jnp.float32),
        name="conv_bn_relu",
        grid=(N,),
        in_specs=[pl.BlockSpec((1, Cin, H8, 8, W), lambda g: (g, 0, 0, 0, 0)),
                  pl.BlockSpec((3, Cout, 3 * Cin), lambda g: (0, 0, 0)),
                  pl.BlockSpec((Cout, 1), lambda g: (0, 0)),
                  pl.BlockSpec((N, Cout, 2), lambda g: (0, 0, 0)),
                  pl.BlockSpec((Cout, 1), lambda g: (0, 0)),
                  pl.BlockSpec((Cout, 1), lambda g: (0, 0))],
        out_specs=pl.BlockSpec((1, Cout, H8, 8, W), lambda g: (g, 0, 0, 0, 0)),
        scratch_shapes=[pltpu.VMEM((3 * Cin, HPW), jnp.bfloat16)],
        compiler_params=pltpu.CompilerParams(
            dimension_semantics=("parallel",),
            vmem_limit_bytes=64 * 1024 * 1024),
    )(xf, w_r, b2, st, g2, be2)

    return out.reshape(N, Cout, H, W)
